# Initial kernel scaffold; baseline (speedup 1.0000x reference)
import jax
import jax.numpy as jnp
from jax.experimental import pallas as pl


def kernel(x, edge_index, edge_weights, W, b):
    raise NotImplementedError("write your pallas kernel here")


# speedup vs baseline: 10.5925x; 10.5925x over previous
import functools

import jax
import jax.numpy as jnp
from jax import lax
from jax.experimental import pallas as pl
from jax.experimental.pallas import tpu as pltpu
from jax.experimental.pallas import tpu_sc as plsc

NC = 2
NS = 16
NW = NC * NS
L = 16
BLK = 128

N = 10000
D = 128
N_PAD = 10240
NPT = N_PAD // NS
ROW_BLK = 1000


def _mesh():
    return plsc.VectorSubcoreMesh(core_axis_name="c", subcore_axis_name="s")


def _deg_kernel(nblk_pt):
    @functools.partial(
        pl.kernel,
        out_type=[jax.ShapeDtypeStruct((N_PAD,), jnp.float32),
                  jax.ShapeDtypeStruct((N_PAD,), jnp.float32)],
        mesh=_mesh(),
        scratch_types=[
            pltpu.VMEM((nblk_pt, BLK), jnp.int32),
            pltpu.VMEM((nblk_pt, BLK), jnp.float32),
            pltpu.VMEM_SHARED((N_PAD,), jnp.float32),
        ],
    )
    def deg_k(colr, ewr, zeros1, deg0, deg1, col_v, ew_v, deg_sp):
        cid = lax.axis_index("c")
        sid = lax.axis_index("s")
        wid = sid * NC + cid
        pltpu.sync_copy(zeros1.at[pl.ds(sid * 640, 640)],
                        deg_sp.at[pl.ds(sid * 640, 640)])
        pltpu.sync_copy(colr.at[pl.ds(wid * nblk_pt, nblk_pt), :], col_v)
        pltpu.sync_copy(ewr.at[pl.ds(wid * nblk_pt, nblk_pt), :], ew_v)
        plsc.subcore_barrier()

        def body(j, carry):
            pltpu.sync_copy(ew_v.at[j], deg_sp.at[col_v.at[j]], add=True)
            return carry

        lax.fori_loop(0, nblk_pt, body, jnp.int32(0))
        plsc.subcore_barrier()

        @pl.when(cid == 0)
        def _():
            pltpu.sync_copy(deg_sp.at[pl.ds(sid * 640, 640)],
                            deg0.at[pl.ds(sid * 640, 640)])

        @pl.when(cid == 1)
        def _():
            pltpu.sync_copy(deg_sp.at[pl.ds(sid * 640, 640)],
                            deg1.at[pl.ds(sid * 640, 640)])

    return deg_k


def _agg_kernel(nblk_pt):
    @functools.partial(
        pl.kernel,
        out_type=jax.ShapeDtypeStruct((NC, N_PAD, D), jnp.float32),
        mesh=_mesh(),
        scratch_types=[
            pltpu.VMEM((4, 3, BLK), jnp.int32),
            pltpu.VMEM((BLK, D), jnp.float32),
            pltpu.VMEM((BLK, D), jnp.float32),
            pltpu.VMEM_SHARED((N_PAD, D), jnp.float32),
            [pltpu.SemaphoreType.DMA] * 4,
            [pltpu.SemaphoreType.DMA] * 2,
        ],
    )
    def agg_k(edges, z, zeros2, accp, ering, zb0, zb1, acc_sp, esems, gsems):
        cid = lax.axis_index("c")
        sid = lax.axis_index("s")
        wid = sid * NC + cid
        base = wid * nblk_pt
        zbs = (zb0, zb1)
        pltpu.sync_copy(zeros2, acc_sp.at[pl.ds(sid * NPT, NPT), :])
        for b in range(4):
            pltpu.async_copy(edges.at[base + b], ering.at[b], esems[b])
        plsc.subcore_barrier()
        for b in range(2):
            pltpu.make_async_copy(edges.at[base + b], ering.at[b],
                                  esems[b]).wait()
            pltpu.async_copy(z.at[ering.at[b, 0]], zbs[b], gsems[b])

        def quad(jj, carry):
            j0 = base + jj * 4
            for b in range(4):
                j = j0 + b
                zb = zbs[b % 2]
                gsem = gsems[b % 2]
                pltpu.make_async_copy(z.at[ering.at[b, 0]], zb, gsem).wait()
                for g in range(BLK // L):
                    cvec = lax.bitcast_convert_type(
                        ering[b, 2, pl.ds(g * L, L)], jnp.float32)
                    for t in range(L):
                        e = g * L + t
                        cv = jnp.full((L,), cvec[t])
                        for k in range(D // L):
                            zb[e, pl.ds(k * L, L)] = zb[e, pl.ds(k * L, L)] * cv
                pltpu.sync_copy(zb, acc_sp.at[ering.at[b, 1]], add=True)

                @pl.when(j + 4 < base + nblk_pt)
                def _():
                    pltpu.async_copy(edges.at[j + 4], ering.at[b], esems[b])

                @pl.when(j + 2 < base + nblk_pt)
                def _():
                    b2 = (b + 2) % 4
                    pltpu.make_async_copy(edges.at[j + 2], ering.at[b2],
                                          esems[b2]).wait()
                    pltpu.async_copy(z.at[ering.at[b2, 0]], zb, gsem)
            return carry

        lax.fori_loop(0, nblk_pt // 4, quad, jnp.int32(0))
        plsc.subcore_barrier()
        pltpu.sync_copy(acc_sp.at[pl.ds(sid * NPT, NPT), :],
                        accp.at[cid, pl.ds(sid * NPT, NPT), :])

    return agg_k


def _z_body(x_ref, w_ref, degp_ref, z_ref):
    x = x_ref[:, :]
    s = jnp.sum(x * x, axis=1, keepdims=True)
    xn = x * lax.rsqrt(jnp.maximum(s, 1e-24))
    deg = degp_ref[0] + degp_ref[1]
    dis = jnp.where(deg > 0, lax.rsqrt(deg), 0.0)
    y = lax.dot_general(xn, w_ref[:, :], (((1,), (1,)), ((), ())),
                        preferred_element_type=jnp.float32)
    z_ref[:, :] = y * dis


def _out_body(accp_ref, degp_ref, b_ref, out_ref):
    acc = accp_ref[0] + accp_ref[1]
    deg = degp_ref[0] + degp_ref[1]
    dis = jnp.where(deg > 0, lax.rsqrt(deg), 0.0)
    out_ref[:, :] = acc * dis + b_ref[:, :]


def kernel(x, edge_index, edge_weights, W, b):
    n, d = x.shape
    e = edge_index.shape[1]
    assert n == N and d == D

    nblk_total = -(-e // BLK)
    nblk_pt = -(-nblk_total // NW)
    nblk_pt += nblk_pt % 2
    e_pad = nblk_pt * NW * BLK

    row = edge_index[0]
    col = edge_index[1]
    pad = e_pad - e
    rowp = jnp.concatenate([row, jnp.zeros((pad,), row.dtype)]).reshape(-1, BLK)
    colp = jnp.concatenate([col, jnp.zeros((pad,), col.dtype)]).reshape(-1, BLK)
    ewp = jnp.concatenate(
        [edge_weights, jnp.zeros((pad,), edge_weights.dtype)]).reshape(-1, BLK)
    edges_packed = jnp.stack(
        [rowp, colp, lax.bitcast_convert_type(ewp, jnp.int32)], axis=1)
    zeros1 = jnp.zeros((N_PAD,), jnp.float32)
    zeros2 = jnp.zeros((NPT, D), jnp.float32)

    deg0, deg1 = _deg_kernel(nblk_pt)(colp, ewp, zeros1)
    degp_n = jnp.stack([deg0, deg1]).reshape(NC, N_PAD, 1)

    grid = N // ROW_BLK
    z = pl.pallas_call(
        _z_body,
        grid=(grid,),
        in_specs=[
            pl.BlockSpec((ROW_BLK, D), lambda i: (i, 0)),
            pl.BlockSpec((D, D), lambda i: (0, 0)),
            pl.BlockSpec((NC, ROW_BLK, 1), lambda i: (0, i, 0)),
        ],
        out_specs=pl.BlockSpec((ROW_BLK, D), lambda i: (i, 0)),
        out_shape=jax.ShapeDtypeStruct((N, D), jnp.float32),
    )(x, W, degp_n)

    accp = _agg_kernel(nblk_pt)(edges_packed, z, zeros2)

    out = pl.pallas_call(
        _out_body,
        grid=(grid,),
        in_specs=[
            pl.BlockSpec((NC, ROW_BLK, D), lambda i: (0, i, 0)),
            pl.BlockSpec((NC, ROW_BLK, 1), lambda i: (0, i, 0)),
            pl.BlockSpec((1, D), lambda i: (0, 0)),
        ],
        out_specs=pl.BlockSpec((ROW_BLK, D), lambda i: (i, 0)),
        out_shape=jax.ShapeDtypeStruct((N, D), jnp.float32),
    )(accp, degp_n, b.reshape(1, D))
    return out
